# baseline (speedup 1.0000x reference)
import functools

import jax
import jax.numpy as jnp
from jax import lax
from jax.experimental import pallas as pl
from jax.experimental.pallas import tpu as pltpu
from jax.experimental.pallas import tpu_sc as plsc

_B, _A, _C = 32, 20000, 21
_NEG_RATIO = 3
_INF_PAT = 0x7F800000


def _ce_kernel(x_ref, lbl_ref, ce_ref):
    x = x_ref[0]
    lbl = lbl_ref[0]
    m = jnp.max(x, axis=0, keepdims=True)
    s = jnp.sum(jnp.exp(x - m), axis=0, keepdims=True)
    cls_iota = lax.broadcasted_iota(jnp.int32, (_C, _A), 0)
    picked = jnp.sum(jnp.where(cls_iota == lbl, x, 0.0),
                     axis=0, keepdims=True)
    ce_ref[0] = jnp.log(s) + m - picked


_SC_CH = 4000


def _sc_loc_body(ploc_hbm, gloc_hbm, lbl_hbm, locout_hbm,
                 lblv, plocv, glocv, outv):
    wid = lax.axis_index("s") * 2 + lax.axis_index("c")
    quad = lax.shift_right_logical(
        lax.broadcasted_iota(jnp.int32, (16,), 0), 2)
    loc_acc = jnp.zeros((16,), jnp.float32)
    for c in range(_A // _SC_CH):
        pltpu.sync_copy(
            lbl_hbm.at[pl.ds(wid * _A + c * _SC_CH, _SC_CH)], lblv)
        pltpu.sync_copy(
            ploc_hbm.at[pl.ds(wid * _A * 4 + c * _SC_CH * 4, _SC_CH * 4)],
            plocv)
        pltpu.sync_copy(
            gloc_hbm.at[pl.ds(wid * _A * 4 + c * _SC_CH * 4, _SC_CH * 4)],
            glocv)

        def group_body(i, acc):
            lv = lblv[pl.ds(i * 16, 16)]
            pos16 = jnp.where(lv > 0, 1.0, 0.0)
            for q in range(4):
                pv = plocv[pl.ds((i * 4 + q) * 16, 16)]
                gv = glocv[pl.ds((i * 4 + q) * 16, 16)]
                d = jnp.abs(pv - gv)
                s = jnp.where(d < 1.0, 0.5 * d * d, d - 0.5)
                pw = lax.gather(
                    pos16, (q * 4 + quad)[:, None],
                    lax.GatherDimensionNumbers(
                        offset_dims=(), collapsed_slice_dims=(0,),
                        start_index_map=(0,)),
                    slice_sizes=(1,),
                    mode=lax.GatherScatterMode.PROMISE_IN_BOUNDS)
                acc = acc + s * pw
            return acc

        loc_acc = lax.fori_loop(0, _SC_CH // 16, group_body, loc_acc)
    outv[...] = loc_acc
    pltpu.sync_copy(outv, locout_hbm.at[pl.ds(wid * 16, 16)])


def _sc_loc_call(ploc_flat, gloc_flat, gt_label):
    mesh = plsc.VectorSubcoreMesh(core_axis_name="c", subcore_axis_name="s")
    fn = functools.partial(
        pl.kernel,
        mesh=mesh,
        out_type=jax.ShapeDtypeStruct((_B * 16,), jnp.float32),
        scratch_types=[
            pltpu.VMEM((_SC_CH,), jnp.int32),
            pltpu.VMEM((_SC_CH * 4,), jnp.float32),
            pltpu.VMEM((_SC_CH * 4,), jnp.float32),
            pltpu.VMEM((16,), jnp.float32),
        ],
    )(_sc_loc_body)
    return fn(ploc_flat, gloc_flat, gt_label)


def _select_kernel(ce_ref, gl_ref, cls_ref, npos_ref):
    ce = ce_ref[...]
    lbl = gl_ref[...]
    pos = lbl > 0
    npos = jnp.sum(pos.astype(jnp.int32), axis=(1, 2), keepdims=True)
    k = jnp.minimum(_NEG_RATIO * npos, _A - 1)

    pt = jnp.exp(-ce)
    one_m_pt = 1.0 - pt
    focal = one_m_pt * one_m_pt * ce
    losses = jnp.where(pos, 0.0, focal)
    lv = lax.bitcast_convert_type(losses, jnp.int32)

    def body(_, carry):
        lo, hi = carry
        mid = lo + (hi - lo) // 2
        cnt = jnp.sum((lv > mid).astype(jnp.int32), axis=(1, 2),
                      keepdims=True)
        ge = cnt >= k
        return jnp.where(ge, mid, lo), jnp.where(ge, hi, mid)

    lo0 = jnp.full((_B, 1, 1), -1, jnp.int32)
    hi0 = jnp.full((_B, 1, 1), _INF_PAT, jnp.int32)
    _, t_pat = lax.fori_loop(0, 31, body, (lo0, hi0))

    gt_t = lv > t_pat
    cnt_gt = jnp.sum(gt_t.astype(jnp.int32), axis=(1, 2), keepdims=True)
    remaining = (k - cnt_gt).astype(jnp.float32)

    eq = (lv == t_pat).astype(jnp.float32)
    lane_iota = lax.broadcasted_iota(jnp.int32, (_B, 8, 2500), 2)
    scan = eq
    for sh in (1, 2, 4, 8, 16, 32, 64, 128, 256, 512, 1024, 2048):
        rolled = pltpu.roll(scan, sh, axis=2)
        scan = scan + jnp.where(lane_iota >= sh, rolled, 0.0)
    row_tot = jnp.sum(eq, axis=2, keepdims=True)
    sub_iota = lax.broadcasted_iota(jnp.int32, (_B, 8, 1), 1)
    row_incl = row_tot
    for sh in (1, 2, 4):
        rolled = pltpu.roll(row_incl, sh, axis=1)
        row_incl = row_incl + jnp.where(sub_iota >= sh, rolled, 0.0)
    row_off = row_incl - row_tot
    c_incl = scan + row_off
    tie_sel = jnp.logical_and(eq > 0.0, c_incl <= remaining)

    mask = jnp.logical_or(pos, jnp.logical_or(gt_t, tie_sel))
    cls_ref[...] = jnp.sum(ce * mask.astype(jnp.float32), axis=(1, 2),
                           keepdims=True)
    npos_ref[...] = npos.astype(jnp.float32)


@jax.jit
def kernel(pred_loc, pred_label, gt_loc, gt_label):
    B, A, C = _B, _A, _C
    loc_part = _sc_loc_call(pred_loc.reshape(B * A * 4),
                            gt_loc.reshape(B * A * 4),
                            gt_label.reshape(B * A))

    xt = jnp.swapaxes(pred_label, 1, 2)
    lbl_row = gt_label.reshape(B, 1, A)
    ce = pl.pallas_call(
        _ce_kernel,
        grid=(B,),
        in_specs=[
            pl.BlockSpec((1, C, A), lambda b: (b, 0, 0)),
            pl.BlockSpec((1, 1, A), lambda b: (b, 0, 0)),
        ],
        out_specs=pl.BlockSpec((1, 1, A), lambda b: (b, 0, 0)),
        out_shape=jax.ShapeDtypeStruct((B, 1, A), jnp.float32),
    )(xt, lbl_row)

    ce_r = ce.reshape(B, 8, A // 8)
    gl_r = gt_label.reshape(B, 8, A // 8)
    cls_part, npos = pl.pallas_call(
        _select_kernel,
        in_specs=[
            pl.BlockSpec((B, 8, A // 8), lambda: (0, 0, 0)),
            pl.BlockSpec((B, 8, A // 8), lambda: (0, 0, 0)),
        ],
        out_specs=[
            pl.BlockSpec((B, 1, 1), lambda: (0, 0, 0)),
            pl.BlockSpec((B, 1, 1), lambda: (0, 0, 0)),
        ],
        out_shape=[
            jax.ShapeDtypeStruct((B, 1, 1), jnp.float32),
            jax.ShapeDtypeStruct((B, 1, 1), jnp.float32),
        ],
    )(ce_r, gl_r)

    n = jnp.sum(npos)
    return (jnp.sum(loc_part) / n, jnp.sum(cls_part) / n)


# speedup vs baseline: 5.9980x; 5.9980x over previous
import jax
import jax.numpy as jnp
from jax import lax
from jax.experimental import pallas as pl
from jax.experimental.pallas import tpu as pltpu

_B, _A, _C = 32, 20000, 21
_NEG_RATIO = 3
_INF_PAT = 0x7F800000


def _ce_loc_kernel(x_ref, lbl_ref, ploc_ref, gloc_ref, gl4_ref,
                   ce_ref, loc_ref):
    x = x_ref[0]
    lbl = lbl_ref[0]
    m = jnp.max(x, axis=0, keepdims=True)
    s = jnp.sum(jnp.exp(x - m), axis=0, keepdims=True)
    cls_iota = lax.broadcasted_iota(jnp.int32, (_C, _A), 0)
    picked = jnp.sum(jnp.where(cls_iota == lbl, x, 0.0),
                     axis=0, keepdims=True)
    ce_ref[0] = jnp.log(s) + m - picked

    d = jnp.abs(ploc_ref[0] - gloc_ref[0])
    sl1 = jnp.where(d < 1.0, 0.5 * d * d, d - 0.5)
    pos4 = (gl4_ref[0] > 0).astype(jnp.float32)
    loc_ref[...] = jnp.sum(sl1 * pos4).reshape(1, 1, 1)


def _select_kernel(ce_ref, gl_ref, cls_ref, npos_ref):
    ce = ce_ref[...]
    lbl = gl_ref[...]
    pos = lbl > 0
    npos = jnp.sum(pos.astype(jnp.int32), axis=(1, 2), keepdims=True)
    k = jnp.minimum(_NEG_RATIO * npos, _A - 1)

    pt = jnp.exp(-ce)
    one_m_pt = 1.0 - pt
    focal = one_m_pt * one_m_pt * ce
    losses = jnp.where(pos, 0.0, focal)
    lv = lax.bitcast_convert_type(losses, jnp.int32)

    def body(_, carry):
        lo, hi = carry
        mid = lo + (hi - lo) // 2
        cnt = jnp.sum((lv > mid).astype(jnp.int32), axis=(1, 2),
                      keepdims=True)
        ge = cnt >= k
        return jnp.where(ge, mid, lo), jnp.where(ge, hi, mid)

    lo0 = jnp.full((_B, 1, 1), -1, jnp.int32)
    hi0 = jnp.full((_B, 1, 1), _INF_PAT, jnp.int32)
    _, t_pat = lax.fori_loop(0, 31, body, (lo0, hi0))

    gt_t = lv > t_pat
    cnt_gt = jnp.sum(gt_t.astype(jnp.int32), axis=(1, 2), keepdims=True)
    remaining = (k - cnt_gt).astype(jnp.float32)

    eq = (lv == t_pat).astype(jnp.float32)
    lane_iota = lax.broadcasted_iota(jnp.int32, (_B, 8, 2500), 2)
    scan = eq
    for sh in (1, 2, 4, 8, 16, 32, 64, 128, 256, 512, 1024, 2048):
        rolled = pltpu.roll(scan, sh, axis=2)
        scan = scan + jnp.where(lane_iota >= sh, rolled, 0.0)
    row_tot = jnp.sum(eq, axis=2, keepdims=True)
    sub_iota = lax.broadcasted_iota(jnp.int32, (_B, 8, 1), 1)
    row_incl = row_tot
    for sh in (1, 2, 4):
        rolled = pltpu.roll(row_incl, sh, axis=1)
        row_incl = row_incl + jnp.where(sub_iota >= sh, rolled, 0.0)
    row_off = row_incl - row_tot
    c_incl = scan + row_off
    tie_sel = jnp.logical_and(eq > 0.0, c_incl <= remaining)

    mask = jnp.logical_or(pos, jnp.logical_or(gt_t, tie_sel))
    cls_ref[...] = jnp.sum(ce * mask.astype(jnp.float32), axis=(1, 2),
                           keepdims=True)
    npos_ref[...] = npos.astype(jnp.float32)


@jax.jit
def kernel(pred_loc, pred_label, gt_loc, gt_label):
    B, A, C = _B, _A, _C
    xt = jnp.swapaxes(pred_label, 1, 2)
    lbl_row = gt_label.reshape(B, 1, A)
    ploc = pred_loc.reshape(B, 8, A * 4 // 8)
    gloc = gt_loc.reshape(B, 8, A * 4 // 8)
    gl4 = jnp.repeat(gt_label, 4, axis=-1).reshape(B, 8, A * 4 // 8)

    ce, loc_part = pl.pallas_call(
        _ce_loc_kernel,
        grid=(B,),
        in_specs=[
            pl.BlockSpec((1, C, A), lambda b: (b, 0, 0)),
            pl.BlockSpec((1, 1, A), lambda b: (b, 0, 0)),
            pl.BlockSpec((1, 8, A * 4 // 8), lambda b: (b, 0, 0)),
            pl.BlockSpec((1, 8, A * 4 // 8), lambda b: (b, 0, 0)),
            pl.BlockSpec((1, 8, A * 4 // 8), lambda b: (b, 0, 0)),
        ],
        out_specs=[
            pl.BlockSpec((1, 1, A), lambda b: (b, 0, 0)),
            pl.BlockSpec((1, 1, 1), lambda b: (b, 0, 0)),
        ],
        out_shape=[
            jax.ShapeDtypeStruct((B, 1, A), jnp.float32),
            jax.ShapeDtypeStruct((B, 1, 1), jnp.float32),
        ],
    )(xt, lbl_row, ploc, gloc, gl4)

    ce_r = ce.reshape(B, 8, A // 8)
    gl_r = gt_label.reshape(B, 8, A // 8)
    cls_part, npos = pl.pallas_call(
        _select_kernel,
        in_specs=[
            pl.BlockSpec((B, 8, A // 8), lambda: (0, 0, 0)),
            pl.BlockSpec((B, 8, A // 8), lambda: (0, 0, 0)),
        ],
        out_specs=[
            pl.BlockSpec((B, 1, 1), lambda: (0, 0, 0)),
            pl.BlockSpec((B, 1, 1), lambda: (0, 0, 0)),
        ],
        out_shape=[
            jax.ShapeDtypeStruct((B, 1, 1), jnp.float32),
            jax.ShapeDtypeStruct((B, 1, 1), jnp.float32),
        ],
    )(ce_r, gl_r)

    n = jnp.sum(npos)
    return (jnp.sum(loc_part) / n, jnp.sum(cls_part) / n)
